# Initial kernel scaffold; baseline (speedup 1.0000x reference)
import jax
import jax.numpy as jnp
from jax.experimental import pallas as pl


def kernel(inputs, W_up, W_gate_proj, W_down, W_router, up_A, up_B, down_A, down_B, gate_A, gate_B):
    raise NotImplementedError("write your pallas kernel here")


# speedup vs baseline: 27.5048x; 27.5048x over previous
import functools

import jax
import jax.numpy as jnp
from jax import lax
from jax.experimental import pallas as pl
from jax.experimental.pallas import tpu as pltpu

E = 64
RANK = 16
D_MODEL = 1024
D_FF = 2048
ALPHA = 2.0
S = 2048

ROW_TILE = 256


def _router_body(x_ref, wr_ref, sel_ref, coef_ref):
    x = x_ref[...]
    wr = wr_ref[...]
    logits = lax.dot_general(x, wr, (((1,), (1,)), ((), ())),
                             preferred_element_type=jnp.float32)
    m = jnp.max(logits, axis=1, keepdims=True)
    eids = lax.broadcasted_iota(jnp.int32, logits.shape, 1)
    sel = jnp.min(jnp.where(logits >= m, eids, E), axis=1, keepdims=True)
    sel_ref[...] = sel
    z = m - jnp.max(m)
    p = jnp.exp(z)
    coef_ref[...] = p / jnp.sum(p)


def _main_body(x_ref, sel_ref, coef_ref, wu_ref, wg_ref, wd_ref,
               au_ref, bu_ref, ag_ref, bg_ref, ad_ref, bd_ref, out_ref):
    xb = x_ref[...]
    sel = sel_ref[...]
    coef = coef_ref[...]

    def mm(a, b):
        return lax.dot_general(a, b, (((1,), (0,)), ((), ())),
                               preferred_element_type=jnp.float32)

    h1 = mm(xb, wu_ref[...])
    h3 = mm(xb, wg_ref[...])
    pu = mm(xb, au_ref[...])
    pg = mm(xb, ag_ref[...])

    col_e = lax.broadcasted_iota(jnp.int32, pu.shape, 1) // RANK
    mask = col_e == sel
    l1 = mm(jnp.where(mask, pu, 0.0).astype(jnp.bfloat16), bu_ref[...])
    l3 = mm(jnp.where(mask, pg, 0.0).astype(jnp.bfloat16), bg_ref[...])

    a = h1 + ALPHA * l1
    b = h3 + ALPHA * l3
    hidden = (a * jax.nn.sigmoid(a) * b).astype(jnp.bfloat16)

    qd = mm(hidden, ad_ref[...])
    l2 = mm(jnp.where(mask, qd, 0.0).astype(jnp.bfloat16), bd_ref[...])
    out_ref[...] = coef * (mm(hidden, wd_ref[...]) + ALPHA * l2)


@jax.jit
def _run(x, W_up, W_gate_proj, W_down, W_router,
         up_A, up_B, down_A, down_B, gate_A, gate_B):
    sel, coef = pl.pallas_call(
        _router_body,
        out_shape=(jax.ShapeDtypeStruct((S, 1), jnp.int32),
                   jax.ShapeDtypeStruct((S, 1), jnp.float32)),
    )(x, W_router)

    bf = jnp.bfloat16
    xb = x.astype(bf)
    wuT = W_up.T.astype(bf)
    wgT = W_gate_proj.T.astype(bf)
    wdT = W_down.T.astype(bf)
    auT = up_A.reshape(E * RANK, D_MODEL).T.astype(bf)
    agT = gate_A.reshape(E * RANK, D_MODEL).T.astype(bf)
    adT = down_A.reshape(E * RANK, D_FF).T.astype(bf)
    bu = up_B.transpose(0, 2, 1).reshape(E * RANK, D_FF).astype(bf)
    bg = gate_B.transpose(0, 2, 1).reshape(E * RANK, D_FF).astype(bf)
    bd = down_B.transpose(0, 2, 1).reshape(E * RANK, D_MODEL).astype(bf)

    n_tiles = S // ROW_TILE
    row = lambda t: (t, 0)
    full = lambda t: (0, 0)
    out = pl.pallas_call(
        _main_body,
        grid=(n_tiles,),
        in_specs=[
            pl.BlockSpec((ROW_TILE, D_MODEL), row),
            pl.BlockSpec((ROW_TILE, 1), row),
            pl.BlockSpec((ROW_TILE, 1), row),
            pl.BlockSpec((D_MODEL, D_FF), full),
            pl.BlockSpec((D_MODEL, D_FF), full),
            pl.BlockSpec((D_FF, D_MODEL), full),
            pl.BlockSpec((D_MODEL, E * RANK), full),
            pl.BlockSpec((E * RANK, D_FF), full),
            pl.BlockSpec((D_MODEL, E * RANK), full),
            pl.BlockSpec((E * RANK, D_FF), full),
            pl.BlockSpec((D_FF, E * RANK), full),
            pl.BlockSpec((E * RANK, D_MODEL), full),
        ],
        out_specs=pl.BlockSpec((ROW_TILE, D_MODEL), row),
        out_shape=jax.ShapeDtypeStruct((S, D_MODEL), jnp.float32),
        compiler_params=pltpu.CompilerParams(
            dimension_semantics=("arbitrary",)),
    )(xb, sel, coef, wuT, wgT, wdT, auT, bu, agT, bg, adT, bd)
    return out


def kernel(inputs, W_up, W_gate_proj, W_down, W_router,
           up_A, up_B, down_A, down_B, gate_A, gate_B):
    x = inputs.reshape(S, D_MODEL)
    out = _run(x, W_up, W_gate_proj, W_down, W_router,
               up_A, up_B, down_A, down_B, gate_A, gate_B)
    return out.reshape(1, S, D_MODEL)
